# baseline (speedup 1.0000x reference)
import functools

import jax
import jax.numpy as jnp
from jax import lax
from jax.experimental import pallas as pl
from jax.experimental.pallas import tpu as pltpu
from jax.experimental.pallas import tpu_sc as plsc

C = 96
HP, WP = 256, 256
H, W = 512, 512
HWP = HP * WP
HW = H * W
N = C * HWP
QN = 8
QSIZE = HW // QN
QSH = 15
WIN = 12288
NC, NS = 2, 16
NWK = NC * NS
TASKS = C * QN
TPW = TASKS // NWK
OFFS_LEN = 784

_mesh = plsc.VectorSubcoreMesh(core_axis_name="c", subcore_axis_name="s")


def _lane_extract(vec, lane):
    sel = lax.broadcasted_iota(jnp.int32, (16,), 0) == lane
    return lax.reduce_max(jnp.where(sel, vec, jnp.int32(-1)), (0,))


@functools.partial(
    pl.kernel,
    out_type=jax.ShapeDtypeStruct((C * HW,), jnp.float32),
    mesh=_mesh,
    scratch_types=[
        pltpu.VMEM((2 * QSIZE,), jnp.float32),
        pltpu.VMEM((WIN + 32,), jnp.int32),
        pltpu.VMEM((WIN,), jnp.float32),
        pltpu.VMEM((OFFS_LEN,), jnp.int32),
        pltpu.VMEM((32,), jnp.int32),
        pltpu.VMEM((16,), jnp.float32),
        pltpu.SemaphoreType.DMA,
        pltpu.SemaphoreType.DMA,
        pltpu.SemaphoreType.DMA,
        pltpu.SemaphoreType.DMA,
    ],
    compiler_params=pltpu.CompilerParams(needs_layout_passes=False),
)
def _unpool_sorted(sk_hbm, sv_hbm, offs_hbm, out_hbm,
                   plane_v, skb, svb, offs_v, tailk, tailv,
                   sem_sk, sem_sv, sem_o0, sem_o1):
    cid = lax.axis_index("c")
    sid = lax.axis_index("s")
    wid = sid * NC + cid

    pltpu.sync_copy(offs_hbm, offs_v)
    pltpu.sync_copy(sk_hbm.at[pl.ds(N - 16, 16)], tailk.at[pl.ds(0, 16)])
    pltpu.sync_copy(sv_hbm.at[pl.ds(N - 16, 16)], tailv)
    tailk[pl.ds(16, 16)] = jnp.full((16,), -1, jnp.int32)

    def out_dma(t, p, sem):
        return pltpu.make_async_copy(
            plane_v.at[pl.ds(p * QSIZE, QSIZE)],
            out_hbm.at[pl.ds(t * QSIZE, QSIZE)], sem)

    def per_task(tl, carry):
        t = wid * TPW + tl
        p = tl & 1
        pbase = p * QSIZE

        chunk_lo = offs_v[pl.ds((t // 16) * 16, 16)]
        s_raw = _lane_extract(chunk_lo, t % 16)
        t1 = t + 1
        chunk_hi = offs_v[pl.ds((t1 // 16) * 16, 16)]
        e_raw = _lane_extract(chunk_hi, t1 % 16)
        s = (s_raw // 16) * 16
        nwin = (e_raw - s + WIN - 1) // WIN

        base0 = pl.multiple_of(jnp.maximum(jnp.minimum(s, N - WIN - 16), 0), 16)
        h_sk = pltpu.async_copy(sk_hbm.at[pl.ds(base0, WIN + 16)],
                                skb.at[pl.ds(0, WIN + 16)], sem_sk)
        h_sv = pltpu.async_copy(sv_hbm.at[pl.ds(base0, WIN)], svb, sem_sv)

        @pl.when(tl >= 2)
        def _():
            @pl.when(p == 0)
            def _():
                out_dma(t, 0, sem_o0).wait()

            @pl.when(p == 1)
            def _():
                out_dma(t, 1, sem_o1).wait()

        def zbody(i, c):
            plane_v[pl.ds(pbase + i * 16, 16)] = jnp.zeros((16,), jnp.float32)
            return c

        lax.fori_loop(0, QSIZE // 16, zbody, 0, unroll=8)

        h_sk.wait()
        h_sv.wait()

        def win_compute(c2):
            def ibody(j, c3):
                a = skb[pl.ds(j * 16, 16)]
                b = skb[pl.ds(j * 16 + 1, 16)]
                v = svb[pl.ds(j * 16, 16)]
                msk = (a != b) & ((a >> QSH) == t)
                plsc.store_scatter(plane_v, [(a & (QSIZE - 1)) + pbase], v,
                                   mask=msk)
                return c3

            return lax.fori_loop(0, WIN // 16, ibody, c2)

        @pl.when(nwin >= 1)
        def _():
            win_compute(0)

        def wbody(w, c):
            base = pl.multiple_of(jnp.maximum(jnp.minimum(s + w * WIN, N - WIN - 16), 0), 16)
            pltpu.sync_copy(sk_hbm.at[pl.ds(base, WIN + 16)],
                            skb.at[pl.ds(0, WIN + 16)])
            pltpu.sync_copy(sv_hbm.at[pl.ds(base, WIN)], svb)
            return win_compute(c)

        lax.fori_loop(1, nwin, wbody, 0)

        a = tailk[pl.ds(0, 16)]
        b = tailk[pl.ds(1, 16)]
        v = tailv[pl.ds(0, 16)]
        msk = (a != b) & ((a >> QSH) == t)
        plsc.store_scatter(plane_v, [(a & (QSIZE - 1)) + pbase], v, mask=msk)

        @pl.when(p == 0)
        def _():
            out_dma(t, 0, sem_o0).start()

        @pl.when(p == 1)
        def _():
            out_dma(t, 1, sem_o1).start()

        return carry

    lax.fori_loop(0, TPW, per_task, 0)

    out_dma(wid * TPW + TPW - 2, 0, sem_o0).wait()
    out_dma(wid * TPW + TPW - 1, 1, sem_o1).wait()


def kernel(x, switches):
    sw = switches.reshape(C, HWP)
    keys = (sw + (jnp.arange(C, dtype=sw.dtype) * HW)[:, None]).reshape(N)
    vals = x.reshape(N)
    sk, sv = lax.sort((keys, vals), num_keys=1, is_stable=False)
    bounds = jnp.arange(TASKS + 1, dtype=jnp.int32) * QSIZE
    offs = jnp.searchsorted(sk, bounds).astype(jnp.int32)
    offs = jnp.concatenate(
        [offs, jnp.full((OFFS_LEN - TASKS - 1,), N, jnp.int32)])
    out = _unpool_sorted(sk, sv, offs)
    return out.reshape(1, C, H, W)


# speedup vs baseline: 1.0227x; 1.0227x over previous
import functools

import jax
import jax.numpy as jnp
from jax import lax
from jax.experimental import pallas as pl
from jax.experimental.pallas import tpu as pltpu
from jax.experimental.pallas import tpu_sc as plsc

C = 96
HP, WP = 256, 256
H, W = 512, 512
HWP = HP * WP
HW = H * W
N = C * HWP
QN = 4
QSIZE = HW // QN
QSH = 16
WIN = 24576
NC, NS = 2, 16
NWK = NC * NS
TASKS = C * QN
TPW = TASKS // NWK
OFFS_LEN = 400

_mesh = plsc.VectorSubcoreMesh(core_axis_name="c", subcore_axis_name="s")


def _lane_extract(vec, lane):
    sel = lax.broadcasted_iota(jnp.int32, (16,), 0) == lane
    return lax.reduce_max(jnp.where(sel, vec, jnp.int32(-1)), (0,))


@functools.partial(
    pl.kernel,
    out_type=jax.ShapeDtypeStruct((C * HW,), jnp.float32),
    mesh=_mesh,
    scratch_types=[
        pltpu.VMEM((QSIZE,), jnp.float32),
        pltpu.VMEM((WIN + 32,), jnp.int32),
        pltpu.VMEM((WIN,), jnp.float32),
        pltpu.VMEM((OFFS_LEN,), jnp.int32),
        pltpu.VMEM((32,), jnp.int32),
        pltpu.VMEM((16,), jnp.float32),
        pltpu.SemaphoreType.DMA,
        pltpu.SemaphoreType.DMA,
    ],
    compiler_params=pltpu.CompilerParams(needs_layout_passes=False),
)
def _unpool_sorted(sk_hbm, sv_hbm, offs_hbm, out_hbm,
                   plane_v, skb, svb, offs_v, tailk, tailv,
                   sem_sk, sem_sv):
    cid = lax.axis_index("c")
    sid = lax.axis_index("s")
    wid = sid * NC + cid

    pltpu.sync_copy(offs_hbm, offs_v)
    pltpu.sync_copy(sk_hbm.at[pl.ds(N - 16, 16)], tailk.at[pl.ds(0, 16)])
    pltpu.sync_copy(sv_hbm.at[pl.ds(N - 16, 16)], tailv)
    tailk[pl.ds(16, 16)] = jnp.full((16,), -1, jnp.int32)

    def per_task(tl, carry):
        t = wid * TPW + tl

        chunk_lo = offs_v[pl.ds((t // 16) * 16, 16)]
        s_raw = _lane_extract(chunk_lo, t % 16)
        t1 = t + 1
        chunk_hi = offs_v[pl.ds((t1 // 16) * 16, 16)]
        e_raw = _lane_extract(chunk_hi, t1 % 16)
        s = (s_raw // 16) * 16
        nwin = (e_raw - s + WIN - 1) // WIN

        base0 = pl.multiple_of(
            jnp.maximum(jnp.minimum(s, N - WIN - 16), 0), 16)
        h_sk = pltpu.async_copy(sk_hbm.at[pl.ds(base0, WIN + 16)],
                                skb.at[pl.ds(0, WIN + 16)], sem_sk)
        h_sv = pltpu.async_copy(sv_hbm.at[pl.ds(base0, WIN)], svb, sem_sv)

        def zbody(i, c):
            plane_v[pl.ds(i * 16, 16)] = jnp.zeros((16,), jnp.float32)
            return c

        lax.fori_loop(0, QSIZE // 16, zbody, 0, unroll=8)

        h_sk.wait()
        h_sv.wait()

        def win_compute(c2):
            def ibody(j, c3):
                a = skb[pl.ds(j * 16, 16)]
                b = skb[pl.ds(j * 16 + 1, 16)]
                v = svb[pl.ds(j * 16, 16)]
                msk = (a != b) & ((a >> QSH) == t)
                plsc.store_scatter(plane_v, [a & (QSIZE - 1)], v, mask=msk)
                return c3

            return lax.fori_loop(0, WIN // 16, ibody, c2)

        @pl.when(nwin >= 1)
        def _():
            win_compute(0)

        def wbody(w, c):
            base = pl.multiple_of(
                jnp.maximum(jnp.minimum(s + w * WIN, N - WIN - 16), 0), 16)
            pltpu.sync_copy(sk_hbm.at[pl.ds(base, WIN + 16)],
                            skb.at[pl.ds(0, WIN + 16)])
            pltpu.sync_copy(sv_hbm.at[pl.ds(base, WIN)], svb)
            return win_compute(c)

        lax.fori_loop(1, nwin, wbody, 0)

        a = tailk[pl.ds(0, 16)]
        b = tailk[pl.ds(1, 16)]
        v = tailv[pl.ds(0, 16)]
        msk = (a != b) & ((a >> QSH) == t)
        plsc.store_scatter(plane_v, [a & (QSIZE - 1)], v, mask=msk)

        pltpu.sync_copy(plane_v, out_hbm.at[pl.ds(t * QSIZE, QSIZE)])
        return carry

    lax.fori_loop(0, TPW, per_task, 0)


def kernel(x, switches):
    sw = switches.reshape(C, HWP)
    keys = (sw + (jnp.arange(C, dtype=sw.dtype) * HW)[:, None]).reshape(N)
    vals = x.reshape(N)
    sk, sv = lax.sort((keys, vals), num_keys=1, is_stable=False)
    bounds = jnp.arange(TASKS + 1, dtype=jnp.int32) * QSIZE
    offs = jnp.searchsorted(sk, bounds).astype(jnp.int32)
    offs = jnp.concatenate(
        [offs, jnp.full((OFFS_LEN - TASKS - 1,), N, jnp.int32)])
    out = _unpool_sorted(sk, sv, offs)
    return out.reshape(1, C, H, W)


# speedup vs baseline: 1.0230x; 1.0003x over previous
import functools

import jax
import jax.numpy as jnp
from jax import lax
from jax.experimental import pallas as pl
from jax.experimental.pallas import tpu as pltpu
from jax.experimental.pallas import tpu_sc as plsc

C = 96
HP, WP = 256, 256
H, W = 512, 512
HWP = HP * WP
HW = H * W
N = C * HWP
QN = 4
QSIZE = HW // QN
QSH = 16
WIN = 24576
NC, NS = 2, 16
NWK = NC * NS
TASKS = C * QN
TPW = TASKS // NWK
OFFS_LEN = 400

_mesh = plsc.VectorSubcoreMesh(core_axis_name="c", subcore_axis_name="s")


def _lane_extract(vec, lane):
    sel = lax.broadcasted_iota(jnp.int32, (16,), 0) == lane
    return lax.reduce_max(jnp.where(sel, vec, jnp.int32(-1)), (0,))


@functools.partial(
    pl.kernel,
    out_type=jax.ShapeDtypeStruct((C * HW,), jnp.float32),
    mesh=_mesh,
    scratch_types=[
        pltpu.VMEM((QSIZE,), jnp.float32),
        pltpu.VMEM((WIN + 32,), jnp.int32),
        pltpu.VMEM((WIN,), jnp.float32),
        pltpu.VMEM((OFFS_LEN,), jnp.int32),
        pltpu.VMEM((32,), jnp.int32),
        pltpu.VMEM((16,), jnp.float32),
        pltpu.SemaphoreType.DMA,
        pltpu.SemaphoreType.DMA,
    ],
    compiler_params=pltpu.CompilerParams(needs_layout_passes=False),
)
def _unpool_sorted(sk_hbm, sv_hbm, offs_hbm, out_hbm,
                   plane_v, skb, svb, offs_v, tailk, tailv,
                   sem_sk, sem_sv):
    cid = lax.axis_index("c")
    sid = lax.axis_index("s")
    wid = sid * NC + cid

    pltpu.sync_copy(offs_hbm, offs_v)
    pltpu.sync_copy(sk_hbm.at[pl.ds(N - 16, 16)], tailk.at[pl.ds(0, 16)])
    pltpu.sync_copy(sv_hbm.at[pl.ds(N - 16, 16)], tailv)
    tailk[pl.ds(16, 16)] = jnp.full((16,), -1, jnp.int32)

    def seg_bounds(t):
        chunk_lo = offs_v[pl.ds((t // 16) * 16, 16)]
        s_raw = _lane_extract(chunk_lo, t % 16)
        t1 = t + 1
        chunk_hi = offs_v[pl.ds((t1 // 16) * 16, 16)]
        e_raw = _lane_extract(chunk_hi, t1 % 16)
        return (s_raw // 16) * 16, e_raw

    def win0_base(s):
        return pl.multiple_of(
            jnp.maximum(jnp.minimum(s, N - WIN - 16), 0), 16)

    def issue_win0(s):
        base0 = win0_base(s)
        pltpu.async_copy(sk_hbm.at[pl.ds(base0, WIN + 16)],
                         skb.at[pl.ds(0, WIN + 16)], sem_sk)
        pltpu.async_copy(sv_hbm.at[pl.ds(base0, WIN)], svb, sem_sv)

    s_first, e_first = seg_bounds(wid * TPW)
    issue_win0(s_first)

    def per_task(tl, carry):
        t = wid * TPW + tl
        s, e_raw = carry
        nwin = (e_raw - s + WIN - 1) // WIN

        def zbody(i, c):
            plane_v[pl.ds(i * 16, 16)] = jnp.zeros((16,), jnp.float32)
            return c

        lax.fori_loop(0, QSIZE // 16, zbody, 0, unroll=8)

        base0 = win0_base(s)
        pltpu.make_async_copy(sk_hbm.at[pl.ds(base0, WIN + 16)],
                              skb.at[pl.ds(0, WIN + 16)], sem_sk).wait()
        pltpu.make_async_copy(sv_hbm.at[pl.ds(base0, WIN)], svb, sem_sv).wait()

        def win_compute(c2):
            def ibody(j, c3):
                a = skb[pl.ds(j * 16, 16)]
                b = skb[pl.ds(j * 16 + 1, 16)]
                v = svb[pl.ds(j * 16, 16)]
                msk = (a != b) & ((a >> QSH) == t)
                plsc.store_scatter(plane_v, [a & (QSIZE - 1)], v, mask=msk)
                return c3

            return lax.fori_loop(0, WIN // 16, ibody, c2)

        @pl.when(nwin >= 1)
        def _():
            win_compute(0)

        def wbody(w, c):
            base = pl.multiple_of(
                jnp.maximum(jnp.minimum(s + w * WIN, N - WIN - 16), 0), 16)
            pltpu.sync_copy(sk_hbm.at[pl.ds(base, WIN + 16)],
                            skb.at[pl.ds(0, WIN + 16)])
            pltpu.sync_copy(sv_hbm.at[pl.ds(base, WIN)], svb)
            return win_compute(c)

        lax.fori_loop(1, nwin, wbody, 0)

        a = tailk[pl.ds(0, 16)]
        b = tailk[pl.ds(1, 16)]
        v = tailv[pl.ds(0, 16)]
        msk = (a != b) & ((a >> QSH) == t)
        plsc.store_scatter(plane_v, [a & (QSIZE - 1)], v, mask=msk)

        s_nxt, e_nxt = seg_bounds(t + 1)
        issue_win0(s_nxt)

        pltpu.sync_copy(plane_v, out_hbm.at[pl.ds(t * QSIZE, QSIZE)])
        return (s_nxt, e_nxt)

    s_last, e_last = lax.fori_loop(0, TPW, per_task, (s_first, e_first))

    base_l = win0_base(s_last)
    pltpu.make_async_copy(sk_hbm.at[pl.ds(base_l, WIN + 16)],
                          skb.at[pl.ds(0, WIN + 16)], sem_sk).wait()
    pltpu.make_async_copy(sv_hbm.at[pl.ds(base_l, WIN)], svb, sem_sv).wait()


def kernel(x, switches):
    sw = switches.reshape(C, HWP)
    keys = (sw + (jnp.arange(C, dtype=sw.dtype) * HW)[:, None]).reshape(N)
    vals = x.reshape(N)
    sk, sv = lax.sort((keys, vals), num_keys=1, is_stable=False)
    bounds = jnp.arange(TASKS + 1, dtype=jnp.int32) * QSIZE
    offs = jnp.searchsorted(sk, bounds).astype(jnp.int32)
    offs = jnp.concatenate(
        [offs, jnp.full((OFFS_LEN - TASKS - 1,), N, jnp.int32)])
    out = _unpool_sorted(sk, sv, offs)
    return out.reshape(1, C, H, W)
